# baseline (speedup 1.0000x reference)
import functools

import jax
import jax.numpy as jnp
from jax.experimental import pallas as pl
from jax.experimental.pallas import tpu as pltpu

_N = 32768
_D = 256
_B = 16
_K = 10
_S = 2
_R = 4096
_NB = _N // (_R * _S)
_BK = _B * _K


def _fused_body(*refs):
    x_refs = refs[0:_S]
    seg_refs = refs[_S:2 * _S]
    (we_ref, be_ref, ww_ref, bw_ref, wq1_ref, bq1_ref, wq2_ref, bq2_ref,
     keys_ref, q1_ref, q2_ref, num_acc, den_acc) = refs[2 * _S:]
    i = pl.program_id(0)

    @pl.when(i == 0)
    def _init():
        num_acc[...] = jnp.zeros_like(num_acc)
        den_acc[...] = jnp.zeros_like(den_acc)

    num = jnp.zeros((_BK + _B, _D), jnp.float32)
    den = jnp.zeros((1, _BK + _B), jnp.float32)
    for s in range(_S):
        x = x_refs[s][...]
        feats = jnp.dot(x, we_ref[...],
                        preferred_element_type=jnp.float32) + be_ref[...]
        w = jax.nn.sigmoid(jnp.dot(feats, ww_ref[...],
                                   preferred_element_type=jnp.float32)
                           + bw_ref[...])

        seg = seg_refs[s][...]
        jcol = jax.lax.broadcasted_iota(jnp.int32, (_R, _BK + _B), 1)
        seg_of_j = jnp.where(jcol < _BK, jcol // _K, jcol - _BK)
        ohext = (seg == seg_of_j).astype(jnp.float32)
        srow = jax.lax.broadcasted_iota(jnp.int32, (_K, _BK + _B), 0)
        scol = jax.lax.broadcasted_iota(jnp.int32, (_K, _BK + _B), 1)
        sel = ((scol % _K == srow) & (scol < _BK)).astype(jnp.float32)
        wext = jnp.dot(w, sel, preferred_element_type=jnp.float32) \
            + (jcol >= _BK).astype(jnp.float32)
        m = ohext * wext

        dn = (((0,), (0,)), ((), ()))
        num = num + jax.lax.dot_general(
            m, feats, dn, preferred_element_type=jnp.float32)
        den = den + jnp.sum(m, axis=0, keepdims=True)

    num_acc[...] += num
    den_acc[...] += den

    @pl.when(i == _NB - 1)
    def _finish():
        dkey = jnp.maximum(den_acc[:, :_BK], 1e-6)
        keys_ref[...] = num_acc[:_BK, :] / dkey.T
        cnt = jnp.maximum(den_acc[:, _BK:], 1.0)
        pooled = num_acc[_BK:, :] / cnt.T
        q1_ref[...] = jnp.dot(pooled, wq1_ref[...],
                              preferred_element_type=jnp.float32) + bq1_ref[...]
        q2_ref[...] = jnp.dot(pooled, wq2_ref[...],
                              preferred_element_type=jnp.float32) + bq2_ref[...]


@functools.partial(jax.jit, static_argnames=("interpret",))
def _run(features, W_enc, b_enc, W_w, b_w, W_q1, b_q1, W_q2, b_q2,
         segment_ids, interpret=False):
    seg2d = segment_ids.astype(jnp.int32).reshape(_N, 1)
    full = lambda shape: pl.BlockSpec(shape, lambda i: (0,) * len(shape))

    def xspec(s):
        return pl.BlockSpec((_R, _D), lambda i, s=s: (i * _S + s, 0))

    def segspec(s):
        return pl.BlockSpec((_R, 1), lambda i, s=s: (i * _S + s, 0))

    keys_flat, q1, q2 = pl.pallas_call(
        _fused_body,
        grid=(_NB,),
        in_specs=(
            [xspec(s) for s in range(_S)]
            + [segspec(s) for s in range(_S)]
            + [
                full((_D, _D)),
                full((1, _D)),
                full((_D, _K)),
                full((1, _K)),
                full((_D, _D)),
                full((1, _D)),
                full((_D, _D)),
                full((1, _D)),
            ]
        ),
        out_specs=[
            full((_BK, _D)),
            full((_B, _D)),
            full((_B, _D)),
        ],
        out_shape=[
            jax.ShapeDtypeStruct((_BK, _D), jnp.float32),
            jax.ShapeDtypeStruct((_B, _D), jnp.float32),
            jax.ShapeDtypeStruct((_B, _D), jnp.float32),
        ],
        scratch_shapes=[
            pltpu.VMEM((_BK + _B, _D), jnp.float32),
            pltpu.VMEM((1, _BK + _B), jnp.float32),
        ],
        interpret=interpret,
    )(*([features] * _S + [seg2d] * _S
        + [W_enc, b_enc.reshape(1, _D), W_w, b_w.reshape(1, _K),
           W_q1, b_q1.reshape(1, _D), W_q2, b_q2.reshape(1, _D)]))
    return keys_flat.reshape(_B, _K, _D), q1, q2


def kernel(features, W_enc, b_enc, W_w, b_w, W_q1, b_q1, W_q2, b_q2,
           segment_ids):
    return _run(features, W_enc, b_enc, W_w, b_w, W_q1, b_q1, W_q2, b_q2,
                segment_ids)


# speedup vs baseline: 1.0917x; 1.0917x over previous
import functools

import jax
import jax.numpy as jnp
from jax.experimental import pallas as pl
from jax.experimental.pallas import tpu as pltpu

_N = 32768
_D = 256
_B = 16
_K = 10
_S = 1
_R = 8192
_NB = _N // (_R * _S)
_BK = _B * _K


def _fused_body(*refs):
    x_refs = refs[0:_S]
    seg_refs = refs[_S:2 * _S]
    (we_ref, be_ref, ww_ref, bw_ref, wq1_ref, bq1_ref, wq2_ref, bq2_ref,
     keys_ref, q1_ref, q2_ref, num_acc, den_acc) = refs[2 * _S:]
    i = pl.program_id(0)

    @pl.when(i == 0)
    def _init():
        num_acc[...] = jnp.zeros_like(num_acc)
        den_acc[...] = jnp.zeros_like(den_acc)

    num = jnp.zeros((_BK + _B, _D), jnp.float32)
    den = jnp.zeros((1, _BK + _B), jnp.float32)
    for s in range(_S):
        x = x_refs[s][...]
        feats = jnp.dot(x, we_ref[...],
                        preferred_element_type=jnp.float32) + be_ref[...]
        w = jax.nn.sigmoid(jnp.dot(feats, ww_ref[...],
                                   preferred_element_type=jnp.float32)
                           + bw_ref[...])

        seg = seg_refs[s][...]
        jcol = jax.lax.broadcasted_iota(jnp.int32, (_R, _BK + _B), 1)
        seg_of_j = jnp.where(jcol < _BK, jcol // _K, jcol - _BK)
        ohext = (seg == seg_of_j).astype(jnp.float32)
        srow = jax.lax.broadcasted_iota(jnp.int32, (_K, _BK + _B), 0)
        scol = jax.lax.broadcasted_iota(jnp.int32, (_K, _BK + _B), 1)
        sel = ((scol % _K == srow) & (scol < _BK)).astype(jnp.float32)
        wext = jnp.dot(w, sel, preferred_element_type=jnp.float32) \
            + (jcol >= _BK).astype(jnp.float32)
        m = ohext * wext

        dn = (((0,), (0,)), ((), ()))
        num = num + jax.lax.dot_general(
            m, feats, dn, preferred_element_type=jnp.float32)
        den = den + jnp.sum(m, axis=0, keepdims=True)

    num_acc[...] += num
    den_acc[...] += den

    @pl.when(i == _NB - 1)
    def _finish():
        dkey = jnp.maximum(den_acc[:, :_BK], 1e-6)
        keys_ref[...] = num_acc[:_BK, :] / dkey.T
        cnt = jnp.maximum(den_acc[:, _BK:], 1.0)
        pooled = num_acc[_BK:, :] / cnt.T
        q1_ref[...] = jnp.dot(pooled, wq1_ref[...],
                              preferred_element_type=jnp.float32) + bq1_ref[...]
        q2_ref[...] = jnp.dot(pooled, wq2_ref[...],
                              preferred_element_type=jnp.float32) + bq2_ref[...]


@functools.partial(jax.jit, static_argnames=("interpret",))
def _run(features, W_enc, b_enc, W_w, b_w, W_q1, b_q1, W_q2, b_q2,
         segment_ids, interpret=False):
    seg2d = segment_ids.astype(jnp.int32).reshape(_N, 1)
    full = lambda shape: pl.BlockSpec(shape, lambda i: (0,) * len(shape))

    def xspec(s):
        return pl.BlockSpec((_R, _D), lambda i, s=s: (i * _S + s, 0))

    def segspec(s):
        return pl.BlockSpec((_R, 1), lambda i, s=s: (i * _S + s, 0))

    keys_flat, q1, q2 = pl.pallas_call(
        _fused_body,
        grid=(_NB,),
        in_specs=(
            [xspec(s) for s in range(_S)]
            + [segspec(s) for s in range(_S)]
            + [
                full((_D, _D)),
                full((1, _D)),
                full((_D, _K)),
                full((1, _K)),
                full((_D, _D)),
                full((1, _D)),
                full((_D, _D)),
                full((1, _D)),
            ]
        ),
        out_specs=[
            full((_BK, _D)),
            full((_B, _D)),
            full((_B, _D)),
        ],
        out_shape=[
            jax.ShapeDtypeStruct((_BK, _D), jnp.float32),
            jax.ShapeDtypeStruct((_B, _D), jnp.float32),
            jax.ShapeDtypeStruct((_B, _D), jnp.float32),
        ],
        scratch_shapes=[
            pltpu.VMEM((_BK + _B, _D), jnp.float32),
            pltpu.VMEM((1, _BK + _B), jnp.float32),
        ],
        interpret=interpret,
    )(*([features] * _S + [seg2d] * _S
        + [W_enc, b_enc.reshape(1, _D), W_w, b_w.reshape(1, _K),
           W_q1, b_q1.reshape(1, _D), W_q2, b_q2.reshape(1, _D)]))
    return keys_flat.reshape(_B, _K, _D), q1, q2


def kernel(features, W_enc, b_enc, W_w, b_w, W_q1, b_q1, W_q2, b_q2,
           segment_ids):
    return _run(features, W_enc, b_enc, W_w, b_w, W_q1, b_q1, W_q2, b_q2,
                segment_ids)
